# baseline (speedup 1.0000x reference)
import jax
import jax.numpy as jnp
from jax.experimental import pallas as pl
from jax.experimental.pallas import tpu as pltpu

WIN = 11
PAD = 5
NH = 4
HD = 8
C = 32
H = 56
W = 56
L = H * W


def _pad2(x, p):
    return jnp.pad(x, ((0, 0), (p, p), (p, p)))


def _conv3(x, w9):
    xp = _pad2(x, 1)
    acc = None
    for t in range(9):
        di, dj = t // 3, t % 3
        xs = xp[:, di:di + H, dj:dj + W].reshape(-1, L)
        r = jnp.dot(w9[t], xs, preferred_element_type=jnp.float32)
        acc = r if acc is None else acc + r
    return acc


def _body(x_ref, w1_ref, bns_ref, bnb_ref, w2_ref, w3_ref, qkvw_ref,
          projw_ref, projb_ref, pb_ref, feat_out_ref, mask_out_ref,
          s_ref, kr_ref, vr_ref, o_ref):
    x = x_ref[0]

    y = _conv3(x, w1_ref[...])
    y = y * bns_ref[...] + bnb_ref[...]
    y = jnp.maximum(y, 0.0)
    y = _conv3(y.reshape(C, H, W), w2_ref[...])
    y = jnp.maximum(y, 0.0)
    y3 = _conv3(y.reshape(C, H, W), w3_ref[...])
    m_sig = jax.nn.sigmoid(y3)
    mask_out_ref[0] = m_sig.reshape(1, H, W)

    mask_f = (m_sig > 0.5).astype(jnp.float32).reshape(H, W)
    mask_low = 1.0 - mask_f.reshape(1, L)

    qkv = jnp.dot(qkvw_ref[...], x.reshape(C, L),
                  preferred_element_type=jnp.float32)
    WR = W + 2 * PAD
    X = H * WR
    EXT = X + 16

    def to_flat(a):
        return jnp.pad(a, ((0, 0), (0, 0), (0, WR - W))).reshape(-1, X)

    q = to_flat(qkv[0:C].reshape(C, H, W))
    kpf = _pad2(qkv[C:2 * C].reshape(C, H, W), PAD).reshape(C, WR * WR)
    vpf = _pad2(qkv[2 * C:3 * C].reshape(C, H, W), PAD).reshape(C, WR * WR)
    wpf = _pad2(mask_f[None], PAD).reshape(1, WR * WR)
    padw = WR * (WIN - 1) + EXT - WR * WR
    kpf = jnp.pad(kpf, ((0, 0), (0, padw)))
    vpf = jnp.pad(vpf, ((0, 0), (0, padw)))
    wpf = jnp.pad(wpf, ((0, 0), (0, padw)))
    kpf = kpf * wpf
    vpf = vpf * wpf
    for r in range(WIN):
        kr_ref[r] = kpf[:, WR * r:WR * r + EXT]
        vr_ref[r] = vpf[:, WR * r:WR * r + EXT]

    r4 = jax.lax.broadcasted_iota(jnp.int32, (NH, C), 0)
    c32 = jax.lax.broadcasted_iota(jnp.int32, (NH, C), 1)
    s4 = (c32 // HD == r4).astype(jnp.float32)
    q = q * (HD ** -0.5)

    SW = 8 * WR
    NS = H // 8

    for t in range(NS):
        base = SW * t

        def logit_row(di, carry, base=base, t=t):
            kseg = kr_ref[di][:, base:base + SW + 16]
            qseg = q[:, base:base + SW]
            pbd = pb_ref[di]
            for dj in range(WIN):
                prod = qseg * kseg[:, dj:dj + SW]
                qk = jnp.dot(s4, prod,
                             preferred_element_type=jnp.float32)
                s_ref[t, di * WIN + dj] = qk + pbd[:, dj][:, None]
            return carry

        jax.lax.fori_loop(0, WIN, logit_row, 0)

        S = s_ref[t]
        mx = S.max(axis=0)
        E = jnp.exp(S - mx[None])
        s_ref[t] = E * (1.0 / E.sum(axis=0))[None]

        def val_row(di, acc, base=base, t=t):
            vseg = vr_ref[di][:, base:base + SW + 16]
            a_row = s_ref[t, pl.ds(di * WIN, WIN)]
            for dj in range(WIN):
                vs = vseg[:, dj:dj + SW].reshape(NH, HD, SW)
                acc = acc + vs * a_row[dj][:, None, :]
            return acc

        o_t = jax.lax.fori_loop(0, WIN, val_row,
                                jnp.zeros((NH, HD, SW), jnp.float32))
        o_ref[:, base:base + SW] = o_t.reshape(C, SW)

    ml_flat = 1.0 - to_flat(mask_f[None].reshape(1, H, W))
    out = o_ref[...] * ml_flat
    fr = jnp.dot(projw_ref[...], out,
                 preferred_element_type=jnp.float32) + projb_ref[...]
    res = to_flat(x) + fr * ml_flat
    feat_out_ref[0] = res.reshape(C, H, WR)[:, :, :W]


def kernel(featL, featR, mh_w1, mh_gamma, mh_beta, mh_mean, mh_var,
           mh_w2, mh_w3, qkv_w, proj_w, proj_b, pos_bias):
    x4 = jnp.concatenate([featL, featR], axis=0)
    bn_scale = (mh_gamma / jnp.sqrt(mh_var + 1e-5)).reshape(C, 1)
    bn_bias = (mh_beta - mh_mean * bn_scale.reshape(C)).reshape(C, 1)
    w1r = mh_w1.transpose(2, 3, 0, 1).reshape(9, C, C)
    w2r = mh_w2.transpose(2, 3, 0, 1).reshape(9, C, C)
    w3r = mh_w3.transpose(2, 3, 0, 1).reshape(9, 1, C)
    qkvw = qkv_w.reshape(3 * C, C)
    projw = proj_w.reshape(C, C)
    projb = proj_b.reshape(C, 1)
    pbr = pos_bias.reshape(NH, WIN, WIN).transpose(1, 0, 2)

    full = lambda shape: pl.BlockSpec(shape, lambda i: (0,) * len(shape))
    feats, masks = pl.pallas_call(
        _body,
        grid=(4,),
        in_specs=[
            pl.BlockSpec((1, C, H, W), lambda i: (i, 0, 0, 0)),
            full((9, C, C)),
            full((C, 1)),
            full((C, 1)),
            full((9, C, C)),
            full((9, 1, C)),
            full((3 * C, C)),
            full((C, C)),
            full((C, 1)),
            full((WIN, NH, WIN)),
        ],
        out_specs=[
            pl.BlockSpec((1, C, H, W), lambda i: (i, 0, 0, 0)),
            pl.BlockSpec((1, 1, H, W), lambda i: (i, 0, 0, 0)),
        ],
        out_shape=[
            jax.ShapeDtypeStruct((4, C, H, W), jnp.float32),
            jax.ShapeDtypeStruct((4, 1, H, W), jnp.float32),
        ],
        scratch_shapes=[
            pltpu.VMEM((H // 8, WIN * WIN, NH, 8 * (W + 2 * PAD)),
                       jnp.float32),
            pltpu.VMEM((WIN, C, H * (W + 2 * PAD) + 16), jnp.float32),
            pltpu.VMEM((WIN, C, H * (W + 2 * PAD) + 16), jnp.float32),
            pltpu.VMEM((C, H * (W + 2 * PAD)), jnp.float32),
        ],
        compiler_params=pltpu.CompilerParams(
            dimension_semantics=("parallel",),
        ),
    )(x4, w1r, bn_scale, bn_bias, w2r, w3r, qkvw, projw, projb, pbr)

    fL, fR = feats[0:2], feats[2:4]
    mL, mR = masks[0:2], masks[2:4]
    return (fL, fR, mL, mR)


# speedup vs baseline: 1.8475x; 1.8475x over previous
import jax
import jax.numpy as jnp
from jax.experimental import pallas as pl
from jax.experimental.pallas import tpu as pltpu

WIN = 11
PAD = 5
NH = 4
HD = 8
C = 32
H = 56
W = 56
L = H * W


def _pad2(x, p):
    return jnp.pad(x, ((0, 0), (p, p), (p, p)))


def _conv3(x, w9):
    xp = _pad2(x, 1)
    acc = None
    for t in range(9):
        di, dj = t // 3, t % 3
        xs = xp[:, di:di + H, dj:dj + W].reshape(-1, L)
        r = jnp.dot(w9[t], xs, preferred_element_type=jnp.float32)
        acc = r if acc is None else acc + r
    return acc


def _body(x_ref, w1_ref, bns_ref, bnb_ref, w2_ref, w3_ref, qkvw_ref,
          projw_ref, projb_ref, pb_ref, feat_out_ref, mask_out_ref,
          s_ref, kr_ref, vr_ref, wr_ref):
    x = x_ref[0]

    y = _conv3(x, w1_ref[...])
    y = y * bns_ref[...] + bnb_ref[...]
    y = jnp.maximum(y, 0.0)
    y = _conv3(y.reshape(C, H, W), w2_ref[...])
    y = jnp.maximum(y, 0.0)
    y3 = _conv3(y.reshape(C, H, W), w3_ref[...])
    m_sig = jax.nn.sigmoid(y3)
    mask_out_ref[0] = m_sig.reshape(1, H, W)

    mask_f = (m_sig > 0.5).astype(jnp.float32).reshape(H, W)
    mask_low = 1.0 - mask_f.reshape(1, L)

    qkv = jnp.dot(qkvw_ref[...], x.reshape(C, L),
                  preferred_element_type=jnp.float32)
    WR = W + 2 * PAD
    X = H * WR
    EXT = X + 16

    def to_flat(a):
        return jnp.pad(a, ((0, 0), (0, 0), (0, WR - W))).reshape(-1, X)

    q = to_flat(qkv[0:C].reshape(C, H, W))
    kpf = _pad2(qkv[C:2 * C].reshape(C, H, W), PAD).reshape(C, WR * WR)
    vpf = _pad2(qkv[2 * C:3 * C].reshape(C, H, W), PAD).reshape(C, WR * WR)
    wpf = _pad2(mask_f[None], PAD).reshape(1, WR * WR)
    padw = WR * (WIN - 1) + EXT - WR * WR
    kpf = jnp.pad(kpf, ((0, 0), (0, padw)))
    vpf = jnp.pad(vpf, ((0, 0), (0, padw)))
    wpf = jnp.pad(wpf, ((0, 0), (0, padw)))
    for r in range(WIN):
        kr_ref[r] = kpf[:, WR * r:WR * r + EXT]
        vr_ref[r] = vpf[:, WR * r:WR * r + EXT]
        wr_ref[r] = wpf[:, WR * r:WR * r + EXT]

    r4 = jax.lax.broadcasted_iota(jnp.int32, (NH, C), 0)
    c32 = jax.lax.broadcasted_iota(jnp.int32, (NH, C), 1)
    s4 = (c32 // HD == r4).astype(jnp.float32)
    scale = HD ** -0.5

    def logit_row(di, carry):
        kdi = kr_ref[di]
        wdi = wr_ref[di]
        pb_row = pb_ref[di]
        rows = []
        for dj in range(WIN):
            prod = q * kdi[:, dj:dj + X]
            qk = jnp.dot(s4, prod,
                         preferred_element_type=jnp.float32)
            ws = wdi[:, dj:dj + X]
            rows.append(qk * (ws * scale) + pb_row[:, dj][:, None])
        s_ref[di] = jnp.stack(rows).reshape(WIN * NH, X)
        return carry

    jax.lax.fori_loop(0, WIN, logit_row, 0)

    S = s_ref[...].reshape(WIN * WIN, NH, X)
    mx = S.max(axis=0)
    E = jnp.exp(S - mx[None])
    attn = E * (1.0 / E.sum(axis=0))[None]
    s_ref[...] = attn.reshape(WIN, WIN * NH, X)

    def val_row(di, acc):
        vdi = vr_ref[di]
        wdi = wr_ref[di]
        a_row = s_ref[di].reshape(WIN, NH, X)
        for dj in range(WIN):
            aw = a_row[dj] * wdi[:, dj:dj + X]
            vs = vdi[:, dj:dj + X].reshape(NH, HD, X)
            acc = acc + vs * aw[:, None, :]
        return acc

    out = jax.lax.fori_loop(0, WIN, val_row,
                            jnp.zeros((NH, HD, X), jnp.float32))

    ml_flat = 1.0 - to_flat(mask_f[None].reshape(1, H, W))
    out = out.reshape(C, X) * ml_flat
    fr = jnp.dot(projw_ref[...], out,
                 preferred_element_type=jnp.float32) + projb_ref[...]
    res = to_flat(x) + fr * ml_flat
    feat_out_ref[0] = res.reshape(C, H, WR)[:, :, :W]


def kernel(featL, featR, mh_w1, mh_gamma, mh_beta, mh_mean, mh_var,
           mh_w2, mh_w3, qkv_w, proj_w, proj_b, pos_bias):
    x4 = jnp.concatenate([featL, featR], axis=0)
    bn_scale = (mh_gamma / jnp.sqrt(mh_var + 1e-5)).reshape(C, 1)
    bn_bias = (mh_beta - mh_mean * bn_scale.reshape(C)).reshape(C, 1)
    w1r = mh_w1.transpose(2, 3, 0, 1).reshape(9, C, C)
    w2r = mh_w2.transpose(2, 3, 0, 1).reshape(9, C, C)
    w3r = mh_w3.transpose(2, 3, 0, 1).reshape(9, 1, C)
    qkvw = qkv_w.reshape(3 * C, C)
    projw = proj_w.reshape(C, C)
    projb = proj_b.reshape(C, 1)
    pbr = pos_bias.reshape(NH, WIN, WIN).transpose(1, 0, 2)

    full = lambda shape: pl.BlockSpec(shape, lambda i: (0,) * len(shape))
    feats, masks = pl.pallas_call(
        _body,
        grid=(4,),
        in_specs=[
            pl.BlockSpec((1, C, H, W), lambda i: (i, 0, 0, 0)),
            full((9, C, C)),
            full((C, 1)),
            full((C, 1)),
            full((9, C, C)),
            full((9, 1, C)),
            full((3 * C, C)),
            full((C, C)),
            full((C, 1)),
            full((WIN, NH, WIN)),
        ],
        out_specs=[
            pl.BlockSpec((1, C, H, W), lambda i: (i, 0, 0, 0)),
            pl.BlockSpec((1, 1, H, W), lambda i: (i, 0, 0, 0)),
        ],
        out_shape=[
            jax.ShapeDtypeStruct((4, C, H, W), jnp.float32),
            jax.ShapeDtypeStruct((4, 1, H, W), jnp.float32),
        ],
        scratch_shapes=[
            pltpu.VMEM((WIN, WIN * NH, H * (W + 2 * PAD)), jnp.float32),
            pltpu.VMEM((WIN, C, H * (W + 2 * PAD) + 16), jnp.float32),
            pltpu.VMEM((WIN, C, H * (W + 2 * PAD) + 16), jnp.float32),
            pltpu.VMEM((WIN, 1, H * (W + 2 * PAD) + 16), jnp.float32),
        ],
        compiler_params=pltpu.CompilerParams(
            dimension_semantics=("parallel",),
        ),
    )(x4, w1r, bn_scale, bn_bias, w2r, w3r, qkvw, projw, projb, pbr)

    fL, fR = feats[0:2], feats[2:4]
    mL, mR = masks[0:2], masks[2:4]
    return (fL, fR, mL, mR)
